# baseline (speedup 1.0000x reference)
import functools

import jax
import jax.numpy as jnp
from jax import lax
from jax.experimental import pallas as pl
from jax.experimental.pallas import tpu as pltpu
from jax.experimental.pallas import tpu_sc as plsc

VOCAB = 100000
CTX = 512
D = 8
DP = 16

_NC, _NS = 2, 16
_NW = _NC * _NS
_TOK_PER_W = CTX // _NW

VT = 2048
_GRID = pl.cdiv(VOCAB, VT)


def _sc_embed(table_p, x, pos_p):
    mesh = plsc.VectorSubcoreMesh(core_axis_name="c", subcore_axis_name="s")

    @functools.partial(
        pl.kernel,
        mesh=mesh,
        out_type=jax.ShapeDtypeStruct((CTX, DP), jnp.float32),
        compiler_params=pltpu.CompilerParams(use_tc_tiling_on_sc=False),
        scratch_types=[
            pltpu.VMEM((_TOK_PER_W,), jnp.int32),
            pltpu.VMEM((_TOK_PER_W, DP), jnp.float32),
            pltpu.VMEM((_TOK_PER_W, DP), jnp.float32),
            pltpu.SemaphoreType.DMA,
        ],
    )
    def k(table_hbm, idx_hbm, pos_hbm, out_hbm, idx_v, rows_v, pos_v, sem):
        wid = lax.axis_index("s") * _NC + lax.axis_index("c")
        base = wid * _TOK_PER_W
        pltpu.sync_copy(idx_hbm.at[pl.ds(base, _TOK_PER_W)], idx_v)
        gather = pltpu.async_copy(table_hbm.at[idx_v], rows_v, sem)
        pltpu.sync_copy(pos_hbm.at[pl.ds(base, _TOK_PER_W)], pos_v)
        gather.wait()
        for i in range(_TOK_PER_W):
            rows_v[i] = rows_v[i] + pos_v[i]
        pltpu.sync_copy(rows_v, out_hbm.at[pl.ds(base, _TOK_PER_W)])

    return k(table_p, x, pos_p)


def _tc_unembed(hidden, w, b2):
    def body(h_ref, w_ref, b_ref, o_ref):
        o_ref[...] = (
            lax.dot_general(
                h_ref[...],
                w_ref[...],
                dimension_numbers=(((1,), (1,)), ((), ())),
                preferred_element_type=jnp.float32,
            )
            + b_ref[...]
        )

    return pl.pallas_call(
        body,
        grid=(_GRID,),
        in_specs=[
            pl.BlockSpec((CTX, D), lambda i: (0, 0)),
            pl.BlockSpec((VT, D), lambda i: (i, 0)),
            pl.BlockSpec((1, VT), lambda i: (0, i)),
        ],
        out_specs=pl.BlockSpec((CTX, VT), lambda i: (0, i)),
        out_shape=jax.ShapeDtypeStruct((CTX, VOCAB), jnp.float32),
    )(hidden, w, b2)


def kernel(x, info_embedding, position_embedding, W, b):
    table_p = jnp.pad(info_embedding, ((0, 0), (0, DP - D)))
    pos_p = jnp.pad(position_embedding, ((0, 0), (0, DP - D)))
    hidden = _sc_embed(table_p, x, pos_p)[:, :D]
    return _tc_unembed(hidden, W, b.reshape(1, VOCAB))


# speedup vs baseline: 1.1323x; 1.1323x over previous
import functools

import jax
import jax.numpy as jnp
from jax import lax
from jax.experimental import pallas as pl
from jax.experimental.pallas import tpu as pltpu
from jax.experimental.pallas import tpu_sc as plsc

VOCAB = 100000
CTX = 512
D = 8
DP = 16

_NC, _NS = 2, 16
_NW = _NC * _NS
_TOK_PER_W = CTX // _NW

VT = 2048
_GRID = pl.cdiv(VOCAB, VT)


def _sc_embed(table_p, x, pos_p):
    mesh = plsc.VectorSubcoreMesh(core_axis_name="c", subcore_axis_name="s")

    @functools.partial(
        pl.kernel,
        mesh=mesh,
        out_type=jax.ShapeDtypeStruct((CTX, DP), jnp.float32),
        compiler_params=pltpu.CompilerParams(use_tc_tiling_on_sc=False),
        scratch_types=[
            pltpu.VMEM((_TOK_PER_W,), jnp.int32),
            pltpu.VMEM((_TOK_PER_W, DP), jnp.float32),
            pltpu.VMEM((_TOK_PER_W, DP), jnp.float32),
            pltpu.SemaphoreType.DMA,
        ],
    )
    def k(table_hbm, idx_hbm, pos_hbm, out_hbm, idx_v, rows_v, pos_v, sem):
        wid = lax.axis_index("s") * _NC + lax.axis_index("c")
        base = wid * _TOK_PER_W
        pltpu.sync_copy(idx_hbm.at[pl.ds(base, _TOK_PER_W)], idx_v)
        gather = pltpu.async_copy(table_hbm.at[idx_v], rows_v, sem)
        pltpu.sync_copy(pos_hbm.at[pl.ds(base, _TOK_PER_W)], pos_v)
        gather.wait()
        for i in range(_TOK_PER_W):
            rows_v[i] = rows_v[i] + pos_v[i]
        pltpu.sync_copy(rows_v, out_hbm.at[pl.ds(base, _TOK_PER_W)])

    return k(table_p, x, pos_p)


TR = 32


def _tc_unembed(hidden, wt, b2):
    def body(h_ref, wt_ref, b_ref, o_ref):
        o_ref[...] = (
            jnp.dot(h_ref[...], wt_ref[...], preferred_element_type=jnp.float32)
            + b_ref[...]
        )

    return pl.pallas_call(
        body,
        grid=(CTX // TR,),
        in_specs=[
            pl.BlockSpec((TR, D), lambda i: (i, 0)),
            pl.BlockSpec((D, VOCAB), lambda i: (0, 0)),
            pl.BlockSpec((1, VOCAB), lambda i: (0, 0)),
        ],
        out_specs=pl.BlockSpec((TR, VOCAB), lambda i: (i, 0)),
        out_shape=jax.ShapeDtypeStruct((CTX, VOCAB), jnp.float32),
    )(hidden, wt, b2)


def kernel(x, info_embedding, position_embedding, W, b):
    table_p = jnp.pad(info_embedding, ((0, 0), (0, DP - D)))
    pos_p = jnp.pad(position_embedding, ((0, 0), (0, DP - D)))
    hidden = _sc_embed(table_p, x, pos_p)[:, :D]
    return _tc_unembed(hidden, W.T, b.reshape(1, VOCAB))
